# Initial kernel scaffold; baseline (speedup 1.0000x reference)
import jax
import jax.numpy as jnp
from jax.experimental import pallas as pl


def kernel(nodes, edges, W_msg, W_un, W_um, W_out):
    raise NotImplementedError("write your pallas kernel here")


# speedup vs baseline: 37.8845x; 37.8845x over previous
import jax
import jax.numpy as jnp
from jax.experimental import pallas as pl
from jax.experimental.pallas import tpu as pltpu

_B, _N = 32, 24
_NODE_F, _HID_F, _EDGE_F, _MSG, _OUT = 128, 128, 16, 128, 128
_PASSES = 3


def _mpnn_body(edges_ref, nodes_ref, w12_ref, w3_ref, wu_ref, wout_ref, out_ref):
    e2 = edges_ref[0]
    adj = jnp.sum(e2, axis=1, keepdims=True)
    edge_act = (adj != 0.0).astype(jnp.float32)
    row_deg = jnp.sum(adj.reshape(_N, _N), axis=1, keepdims=True)
    node_act = row_deg != 0.0

    e3 = jnp.dot(e2, w3_ref[...], preferred_element_type=jnp.float32)
    e3 = e3.reshape(_N, _N, _MSG)

    hidden = nodes_ref[0]
    for _ in range(_PASSES):
        ac = jnp.dot(hidden, w12_ref[...], preferred_element_type=jnp.float32)
        a = ac[:, :_MSG]
        c = ac[:, _MSG:]
        pre = e3 + a[:, None, :] + c[None, :, :]
        mt = jnp.tanh(pre)
        mtm = mt.reshape(_N * _N, _MSG) * edge_act
        msg = jnp.sum(mtm.reshape(_N, _N, _MSG), axis=1)
        hm = jnp.concatenate([hidden, msg], axis=1)
        upd = jnp.tanh(jnp.dot(hm, wu_ref[...], preferred_element_type=jnp.float32))
        hidden = jnp.where(node_act, upd, hidden)

    graph = jnp.sum(jnp.where(node_act, hidden, 0.0), axis=0, keepdims=True)
    out_ref[0] = jnp.dot(graph, wout_ref[...], preferred_element_type=jnp.float32)


def kernel(nodes, edges, W_msg, W_un, W_um, W_out):
    w12 = jnp.concatenate([W_msg[:_HID_F], W_msg[_HID_F : 2 * _HID_F]], axis=1)
    w3 = W_msg[2 * _HID_F :]
    wu = jnp.concatenate([W_un, W_um], axis=0)
    edges_r = edges.reshape(_B, _N * _N, _EDGE_F)

    return pl.pallas_call(
        _mpnn_body,
        grid=(_B,),
        in_specs=[
            pl.BlockSpec((1, _N * _N, _EDGE_F), lambda b: (b, 0, 0)),
            pl.BlockSpec((1, _N, _HID_F), lambda b: (b, 0, 0)),
            pl.BlockSpec((_HID_F, 2 * _MSG), lambda b: (0, 0)),
            pl.BlockSpec((_EDGE_F, _MSG), lambda b: (0, 0)),
            pl.BlockSpec((2 * _HID_F, _HID_F), lambda b: (0, 0)),
            pl.BlockSpec((_HID_F, _OUT), lambda b: (0, 0)),
        ],
        out_specs=pl.BlockSpec((1, 1, _OUT), lambda b: (b, 0, 0)),
        out_shape=jax.ShapeDtypeStruct((_B, 1, _OUT), jnp.float32),
        compiler_params=pltpu.CompilerParams(
            dimension_semantics=("arbitrary",),
        ),
    )(edges_r, nodes, w12, w3, wu, W_out).reshape(_B, _OUT)


# speedup vs baseline: 73.3231x; 1.9354x over previous
import jax
import jax.numpy as jnp
from jax.experimental import pallas as pl
from jax.experimental.pallas import tpu as pltpu

_B, _N = 32, 24
_NODE_F, _HID_F, _EDGE_F, _MSG, _OUT = 128, 128, 16, 128, 128
_PASSES = 3
_BB = 8


def _mpnn_body(edges_ref, nodes_ref, w12_ref, w3_ref, wu_ref, wout_ref, out_ref):
    e2 = edges_ref[0]
    adj = jnp.sum(e2, axis=1, keepdims=True)
    edge_act = (adj != 0.0).astype(jnp.float32)
    row_deg = jnp.sum(adj.reshape(_BB, _N, _N, 1), axis=1)
    node_act = row_deg.reshape(_BB * _N, 1) != 0.0

    e3 = jnp.dot(e2, w3_ref[...], preferred_element_type=jnp.float32)
    e3 = e3.reshape(_BB, _N, _N, _MSG)

    hidden = nodes_ref[0]
    for _ in range(_PASSES):
        ac = jnp.dot(hidden, w12_ref[...], preferred_element_type=jnp.float32)
        a = ac[:, :_MSG].reshape(_BB, 1, _N, _MSG)
        c = ac[:, _MSG:].reshape(_BB, _N, 1, _MSG)
        pre = e3 + a + c
        mt = jnp.tanh(pre.reshape(_BB * _N * _N, _MSG)) * edge_act
        msg = jnp.sum(mt.reshape(_BB, _N, _N, _MSG), axis=1)
        hm = jnp.concatenate([hidden, msg.reshape(_BB * _N, _MSG)], axis=1)
        upd = jnp.tanh(jnp.dot(hm, wu_ref[...], preferred_element_type=jnp.float32))
        hidden = jnp.where(node_act, upd, hidden)

    masked = jnp.where(node_act, hidden, 0.0)
    graph = jnp.sum(masked.reshape(_BB, _N, _HID_F), axis=1)
    out_ref[0] = jnp.dot(graph, wout_ref[...], preferred_element_type=jnp.float32)


def kernel(nodes, edges, W_msg, W_un, W_um, W_out):
    w12 = jnp.concatenate([W_msg[:_HID_F], W_msg[_HID_F : 2 * _HID_F]], axis=1)
    w3 = W_msg[2 * _HID_F :]
    wu = jnp.concatenate([W_un, W_um], axis=0)
    edges_t = edges.transpose(0, 2, 1, 3).reshape(_B // _BB, _BB * _N * _N, _EDGE_F)
    nodes_r = nodes.reshape(_B // _BB, _BB * _N, _HID_F)

    return pl.pallas_call(
        _mpnn_body,
        grid=(_B // _BB,),
        in_specs=[
            pl.BlockSpec((1, _BB * _N * _N, _EDGE_F), lambda b: (b, 0, 0)),
            pl.BlockSpec((1, _BB * _N, _HID_F), lambda b: (b, 0, 0)),
            pl.BlockSpec((_HID_F, 2 * _MSG), lambda b: (0, 0)),
            pl.BlockSpec((_EDGE_F, _MSG), lambda b: (0, 0)),
            pl.BlockSpec((2 * _HID_F, _HID_F), lambda b: (0, 0)),
            pl.BlockSpec((_HID_F, _OUT), lambda b: (0, 0)),
        ],
        out_specs=pl.BlockSpec((1, _BB, _OUT), lambda b: (b, 0, 0)),
        out_shape=jax.ShapeDtypeStruct((_B // _BB, _BB, _OUT), jnp.float32),
        compiler_params=pltpu.CompilerParams(
            dimension_semantics=("arbitrary",),
        ),
    )(edges_t, nodes_r, w12, w3, wu, W_out).reshape(_B, _OUT)


# speedup vs baseline: 74.3472x; 1.0140x over previous
import jax
import jax.numpy as jnp
from jax.experimental import pallas as pl
from jax.experimental.pallas import tpu as pltpu

_B, _N = 32, 24
_NODE_F, _HID_F, _EDGE_F, _MSG, _OUT = 128, 128, 16, 128, 128
_PASSES = 3
_BB = 16


def _mpnn_body(edges_ref, nodes_ref, w12_ref, w3a_ref, wun_ref, wum_ref, wout_ref, out_ref):
    e2 = edges_ref[0]
    e3a = jnp.dot(e2, w3a_ref[...], preferred_element_type=jnp.float32)
    adj = e3a[:, _MSG : _MSG + 1]
    edge_act = (adj != 0.0).astype(jnp.float32)
    e3 = e3a[:, :_MSG].reshape(_BB, _N, _N, _MSG)
    row_deg = jnp.sum(adj.reshape(_BB, _N, _N, 1), axis=1)
    node_act = row_deg.reshape(_BB * _N, 1) != 0.0

    hidden = nodes_ref[0]
    for _ in range(_PASSES):
        ac = jnp.dot(hidden, w12_ref[...], preferred_element_type=jnp.float32)
        a = ac[:, :_MSG].reshape(_BB, 1, _N, _MSG)
        c = ac[:, _MSG:].reshape(_BB, _N, 1, _MSG)
        pre = e3 + a + c
        mt = jnp.tanh(pre.reshape(_BB * _N * _N, _MSG)) * edge_act
        msg = jnp.sum(mt.reshape(_BB, _N, _N, _MSG), axis=1)
        upd = jnp.tanh(
            jnp.dot(hidden, wun_ref[...], preferred_element_type=jnp.float32)
            + jnp.dot(msg.reshape(_BB * _N, _MSG), wum_ref[...], preferred_element_type=jnp.float32)
        )
        hidden = jnp.where(node_act, upd, hidden)

    masked = jnp.where(node_act, hidden, 0.0)
    graph = jnp.sum(masked.reshape(_BB, _N, _HID_F), axis=1)
    out_ref[0] = jnp.dot(graph, wout_ref[...], preferred_element_type=jnp.float32)


def kernel(nodes, edges, W_msg, W_un, W_um, W_out):
    w12 = jnp.concatenate([W_msg[:_HID_F], W_msg[_HID_F : 2 * _HID_F]], axis=1)
    w3 = W_msg[2 * _HID_F :]
    w3a = jnp.concatenate(
        [
            w3,
            jnp.ones((_EDGE_F, 1), jnp.float32),
            jnp.zeros((_EDGE_F, _MSG - 1), jnp.float32),
        ],
        axis=1,
    )
    edges_t = edges.transpose(0, 2, 1, 3).reshape(_B // _BB, _BB * _N * _N, _EDGE_F)
    nodes_r = nodes.reshape(_B // _BB, _BB * _N, _HID_F)

    return pl.pallas_call(
        _mpnn_body,
        grid=(_B // _BB,),
        in_specs=[
            pl.BlockSpec((1, _BB * _N * _N, _EDGE_F), lambda b: (b, 0, 0)),
            pl.BlockSpec((1, _BB * _N, _HID_F), lambda b: (b, 0, 0)),
            pl.BlockSpec((_HID_F, 2 * _MSG), lambda b: (0, 0)),
            pl.BlockSpec((_EDGE_F, 2 * _MSG), lambda b: (0, 0)),
            pl.BlockSpec((_HID_F, _HID_F), lambda b: (0, 0)),
            pl.BlockSpec((_MSG, _HID_F), lambda b: (0, 0)),
            pl.BlockSpec((_HID_F, _OUT), lambda b: (0, 0)),
        ],
        out_specs=pl.BlockSpec((1, _BB, _OUT), lambda b: (b, 0, 0)),
        out_shape=jax.ShapeDtypeStruct((_B // _BB, _BB, _OUT), jnp.float32),
        compiler_params=pltpu.CompilerParams(
            dimension_semantics=("arbitrary",),
        ),
    )(edges_t, nodes_r, w12, w3a, W_un, W_um, W_out).reshape(_B, _OUT)


# speedup vs baseline: 75.3157x; 1.0130x over previous
import jax
import jax.numpy as jnp
from jax.experimental import pallas as pl
from jax.experimental.pallas import tpu as pltpu

_B, _N = 32, 24
_NODE_F, _HID_F, _EDGE_F, _MSG, _OUT = 128, 128, 16, 128, 128
_PASSES = 3
_BB = 16


def _mpnn_body(edges_ref, nodes_ref, wmsg_ref, wun_ref, wum_ref, wout_ref, out_ref):
    e2 = edges_ref[0]
    adj = jnp.sum(e2, axis=1, keepdims=True)
    edge_act = (adj != 0.0).astype(jnp.float32)
    row_deg = jnp.sum(adj.reshape(_BB, _N, _N, 1), axis=2)
    node_act = row_deg.reshape(_BB * _N, 1) != 0.0

    wm = wmsg_ref[...]
    w1 = wm[:_HID_F]
    w2 = wm[_HID_F : 2 * _HID_F]
    w3 = wm[2 * _HID_F :]

    e3 = jnp.dot(e2, w3, preferred_element_type=jnp.float32)
    e3 = e3.reshape(_BB, _N, _N, _MSG)

    hidden = nodes_ref[0]
    for _ in range(_PASSES):
        a = jnp.dot(hidden, w1, preferred_element_type=jnp.float32)
        c = jnp.dot(hidden, w2, preferred_element_type=jnp.float32)
        pre = e3 + a.reshape(_BB, _N, 1, _MSG) + c.reshape(_BB, 1, _N, _MSG)
        mt = jnp.tanh(pre.reshape(_BB * _N * _N, _MSG)) * edge_act
        msg = jnp.sum(mt.reshape(_BB, _N, _N, _MSG), axis=2)
        upd = jnp.tanh(
            jnp.dot(hidden, wun_ref[...], preferred_element_type=jnp.float32)
            + jnp.dot(msg.reshape(_BB * _N, _MSG), wum_ref[...], preferred_element_type=jnp.float32)
        )
        hidden = jnp.where(node_act, upd, hidden)

    masked = jnp.where(node_act, hidden, 0.0)
    graph = jnp.sum(masked.reshape(_BB, _N, _HID_F), axis=1)
    out_ref[0] = jnp.dot(graph, wout_ref[...], preferred_element_type=jnp.float32)


def kernel(nodes, edges, W_msg, W_un, W_um, W_out):
    edges_r = edges.reshape(_B // _BB, _BB * _N * _N, _EDGE_F)
    nodes_r = nodes.reshape(_B // _BB, _BB * _N, _HID_F)

    return pl.pallas_call(
        _mpnn_body,
        grid=(_B // _BB,),
        in_specs=[
            pl.BlockSpec((1, _BB * _N * _N, _EDGE_F), lambda b: (b, 0, 0)),
            pl.BlockSpec((1, _BB * _N, _HID_F), lambda b: (b, 0, 0)),
            pl.BlockSpec((2 * _HID_F + _EDGE_F, _MSG), lambda b: (0, 0)),
            pl.BlockSpec((_HID_F, _HID_F), lambda b: (0, 0)),
            pl.BlockSpec((_MSG, _HID_F), lambda b: (0, 0)),
            pl.BlockSpec((_HID_F, _OUT), lambda b: (0, 0)),
        ],
        out_specs=pl.BlockSpec((1, _BB, _OUT), lambda b: (b, 0, 0)),
        out_shape=jax.ShapeDtypeStruct((_B // _BB, _BB, _OUT), jnp.float32),
        compiler_params=pltpu.CompilerParams(
            dimension_semantics=("arbitrary",),
        ),
    )(edges_r, nodes_r, W_msg, W_un, W_um, W_out).reshape(_B, _OUT)
